# baseline (speedup 1.0000x reference)
import functools

import jax
import jax.numpy as jnp
from jax import lax
from jax.experimental import pallas as pl
from jax.experimental.pallas import tpu as pltpu
from jax.experimental.pallas import tpu_sc as plsc

N_USER = 5000
N_ITEM = 5000
N = N_USER + N_ITEM
E = 320000
D = 128
N_LAYERS = 3

NC = 2
NS = 16
NW = NC * NS
CH = 128
CPW = 80
CA = 160
CB = 2 * CPW - CA
CPB = 8
E_PAD = NW * CPW * CH
N_PAD = 10240
ZRO = N_PAD // NS
EPC = 64


def _layer_body(col_ref, row_ref, val_ref, cur_ref, part_ref,
                col_v, row_v, val_v, gbuf, obuf, acc_sh, sem0, sem1):
    c = lax.axis_index("c")
    s = lax.axis_index("s")
    wid = s * NC + c

    zeros16 = jnp.zeros((16,), jnp.float32)

    def _zero(i, carry):
        for g in range(8):
            obuf[i, pl.ds(g * 16, 16)] = zeros16
        return carry
    lax.fori_loop(0, EPC, _zero, 0)
    for h in range(ZRO // EPC):
        pltpu.sync_copy(obuf, acc_sh.at[pl.ds(s * ZRO + h * EPC, EPC)])

    plsc.subcore_barrier()

    sems = (sem0, sem1)

    def _fire(j, b):
        pltpu.async_copy(cur_ref.at[col_v.at[j]], gbuf.at[b], sems[b])

    def _drain(j, b):
        pltpu.make_async_copy(cur_ref.at[col_v.at[j]], gbuf.at[b],
                              sems[b]).wait()

    def _process(j, b):
        def _grp(g, inner):
            vals16 = val_v[j, pl.ds(g * 16, 16)]
            e0 = g * 16
            for e in range(16):
                v = vals16[e]
                for k in range(D // 16):
                    ssl = pl.ds(k * 16, 16)
                    gbuf[b, e0 + e, ssl] = gbuf[b, e0 + e, ssl] * v
            return inner
        lax.fori_loop(0, CH // 16, _grp, 0)
        pltpu.sync_copy(gbuf.at[b], acc_sh.at[row_v.at[j]], add=True)

    wbase = jnp.where(c == 0, s * CA, NS * CA + s * CB)
    nblk = jnp.where(c == 0, CA // CPB, CB // CPB)

    def _block(blk, bcarry):
        base = wbase + blk * CPB
        pltpu.sync_copy(col_ref.at[pl.ds(base, CPB)], col_v)
        pltpu.sync_copy(row_ref.at[pl.ds(base, CPB)], row_v)
        pltpu.sync_copy(val_ref.at[pl.ds(base, CPB)], val_v)

        _fire(0, 0)

        def _pair(j2, carry):
            j = j2 * 2
            _fire(j + 1, 1)
            _drain(j, 0)
            _process(j, 0)

            @pl.when(j + 2 < CPB)
            def _():
                _fire(j + 2, 0)
            _drain(j + 1, 1)
            _process(j + 1, 1)
            return carry
        lax.fori_loop(0, CPB // 2, _pair, 0)
        return bcarry
    lax.fori_loop(0, nblk, _block, 0)

    plsc.subcore_barrier()

    for h in range(ZRO // EPC):
        r0 = s * ZRO + h * EPC
        pltpu.sync_copy(acc_sh.at[pl.ds(r0, EPC)], obuf)
        pltpu.sync_copy(obuf, part_ref.at[c, pl.ds(r0, EPC)])


def _spmm_layer(col2d, row2d, val2d, cur):
    mesh = plsc.VectorSubcoreMesh(core_axis_name="c", subcore_axis_name="s",
                                  num_cores=NC, num_subcores=NS)
    return pl.kernel(
        _layer_body,
        out_type=jax.ShapeDtypeStruct((NC, N_PAD, D), jnp.float32),
        mesh=mesh,
        scratch_types=[
            pltpu.VMEM((CPB, CH), jnp.int32),
            pltpu.VMEM((CPB, CH), jnp.int32),
            pltpu.VMEM((CPB, CH), jnp.float32),
            pltpu.VMEM((2, CH, D), jnp.float32),
            pltpu.VMEM((EPC, D), jnp.float32),
            pltpu.VMEM_SHARED((N_PAD, D), jnp.float32),
            pltpu.SemaphoreType.DMA,
            pltpu.SemaphoreType.DMA,
        ],
    )(col2d, row2d, val2d, cur)


def _combine_body(p0_ref, p1_ref, acc_ref, cur_out, acc_out):
    cur = p0_ref[0] + p1_ref[0]
    cur_out[...] = cur
    acc_out[...] = acc_ref[...] + cur


def _combine(parts, acc):
    blk = 256
    bs = pl.BlockSpec((blk, D), lambda i: (i, 0))
    bs3 = pl.BlockSpec((1, blk, D), lambda i: (0, i, 0))
    return pl.pallas_call(
        _combine_body,
        grid=(N_PAD // blk,),
        in_specs=[bs3, bs3, bs],
        out_specs=[bs, bs],
        out_shape=[jax.ShapeDtypeStruct((N_PAD, D), jnp.float32),
                   jax.ShapeDtypeStruct((N_PAD, D), jnp.float32)],
    )(parts[0:1], parts[1:2], acc)


def kernel(edge_index, adj_values, uEmbeds, iEmbeds):
    row = edge_index[0].astype(jnp.int32)
    col = edge_index[1].astype(jnp.int32)
    val = adj_values.astype(jnp.float32)
    pad = E_PAD - E
    pad_rows = N + (jnp.arange(pad, dtype=jnp.int32) % (N_PAD - N))
    col2d = jnp.pad(col, (0, pad)).reshape(NW * CPW, CH)
    row2d = jnp.concatenate([row, pad_rows]).reshape(NW * CPW, CH)
    val2d = jnp.pad(val, (0, pad)).reshape(NW * CPW, CH)

    embeds = jnp.concatenate([uEmbeds, iEmbeds], axis=0)
    embeds = jnp.pad(embeds, ((0, N_PAD - N), (0, 0)))
    acc = embeds
    cur = embeds
    for _ in range(N_LAYERS):
        parts = _spmm_layer(col2d, row2d, val2d, cur)
        cur, acc = _combine(parts, acc)
    return acc[:N_USER], acc[N_USER:N]


# speedup vs baseline: 1.5979x; 1.5979x over previous
import functools

import jax
import jax.numpy as jnp
from jax import lax
from jax.experimental import pallas as pl
from jax.experimental.pallas import tpu as pltpu
from jax.experimental.pallas import tpu_sc as plsc

N_USER = 5000
N_ITEM = 5000
N = N_USER + N_ITEM
E = 320000
D = 128
N_LAYERS = 3

NC = 2
NS = 16
NW = NC * NS
CH = 128
CPW = 80
CA = 152
CB = 2 * CPW - CA
CPB = 8
E_PAD = NW * CPW * CH
N_PAD = 10240
ZRO = N_PAD // NS
EPC = 64


def _layer_body(col_ref, row_ref, val_ref, cur_ref, part_ref,
                col_v, row_v, val_v, gbuf, obuf, acc_sh, sem0, sem1):
    c = lax.axis_index("c")
    s = lax.axis_index("s")
    wid = s * NC + c

    zeros16 = jnp.zeros((16,), jnp.float32)

    def _zero(i, carry):
        for g in range(8):
            obuf[i, pl.ds(g * 16, 16)] = zeros16
        return carry
    lax.fori_loop(0, EPC, _zero, 0)
    for h in range(ZRO // EPC):
        pltpu.sync_copy(obuf, acc_sh.at[pl.ds(s * ZRO + h * EPC, EPC)])

    plsc.subcore_barrier()

    sems = (sem0, sem1)

    def _fire(j, b):
        pltpu.async_copy(cur_ref.at[col_v.at[j]], gbuf.at[b], sems[b])

    def _drain(j, b):
        pltpu.make_async_copy(cur_ref.at[col_v.at[j]], gbuf.at[b],
                              sems[b]).wait()

    def _process(j, b):
        def _grp(g, inner):
            vals16 = val_v[j, pl.ds(g * 16, 16)]
            e0 = g * 16
            for e in range(16):
                v = vals16[e]
                for k in range(D // 16):
                    ssl = pl.ds(k * 16, 16)
                    gbuf[b, e0 + e, ssl] = gbuf[b, e0 + e, ssl] * v
            return inner
        lax.fori_loop(0, CH // 16, _grp, 0)
        pltpu.sync_copy(gbuf.at[b], acc_sh.at[row_v.at[j]], add=True)

    wbase = jnp.where(c == 0, s * CA, NS * CA + s * CB)
    nblk = jnp.where(c == 0, CA // CPB, CB // CPB)

    def _block(blk, bcarry):
        base = wbase + blk * CPB
        pltpu.sync_copy(col_ref.at[pl.ds(base, CPB)], col_v)
        pltpu.sync_copy(row_ref.at[pl.ds(base, CPB)], row_v)
        pltpu.sync_copy(val_ref.at[pl.ds(base, CPB)], val_v)

        _fire(0, 0)

        def _pair(j2, carry):
            j = j2 * 2
            _fire(j + 1, 1)
            _drain(j, 0)
            _process(j, 0)

            @pl.when(j + 2 < CPB)
            def _():
                _fire(j + 2, 0)
            _drain(j + 1, 1)
            _process(j + 1, 1)
            return carry
        lax.fori_loop(0, CPB // 2, _pair, 0)
        return bcarry
    lax.fori_loop(0, nblk, _block, 0)

    plsc.subcore_barrier()

    for h in range(ZRO // EPC):
        r0 = s * ZRO + h * EPC
        pltpu.sync_copy(acc_sh.at[pl.ds(r0, EPC)], obuf)
        pltpu.sync_copy(obuf, part_ref.at[c, pl.ds(r0, EPC)])


def _spmm_layer(col2d, row2d, val2d, cur):
    mesh = plsc.VectorSubcoreMesh(core_axis_name="c", subcore_axis_name="s",
                                  num_cores=NC, num_subcores=NS)
    return pl.kernel(
        _layer_body,
        out_type=jax.ShapeDtypeStruct((NC, N_PAD, D), jnp.float32),
        mesh=mesh,
        scratch_types=[
            pltpu.VMEM((CPB, CH), jnp.int32),
            pltpu.VMEM((CPB, CH), jnp.int32),
            pltpu.VMEM((CPB, CH), jnp.float32),
            pltpu.VMEM((2, CH, D), jnp.float32),
            pltpu.VMEM((EPC, D), jnp.float32),
            pltpu.VMEM_SHARED((N_PAD, D), jnp.float32),
            pltpu.SemaphoreType.DMA,
            pltpu.SemaphoreType.DMA,
        ],
    )(col2d, row2d, val2d, cur)


def _combine_body(p0_ref, p1_ref, acc_ref, cur_out, acc_out):
    cur = p0_ref[0] + p1_ref[0]
    cur_out[...] = cur
    acc_out[...] = acc_ref[...] + cur


def _combine(parts, acc):
    blk = 256
    bs = pl.BlockSpec((blk, D), lambda i: (i, 0))
    bs3 = pl.BlockSpec((1, blk, D), lambda i: (0, i, 0))
    return pl.pallas_call(
        _combine_body,
        grid=(N_PAD // blk,),
        in_specs=[bs3, bs3, bs],
        out_specs=[bs, bs],
        out_shape=[jax.ShapeDtypeStruct((N_PAD, D), jnp.float32),
                   jax.ShapeDtypeStruct((N_PAD, D), jnp.float32)],
    )(parts[0:1], parts[1:2], acc)


def kernel(edge_index, adj_values, uEmbeds, iEmbeds):
    row = edge_index[0].astype(jnp.int32)
    col = edge_index[1].astype(jnp.int32)
    val = adj_values.astype(jnp.float32)
    pad = E_PAD - E
    pad_rows = N + (jnp.arange(pad, dtype=jnp.int32) % (N_PAD - N))
    col2d = jnp.pad(col, (0, pad)).reshape(NW * CPW, CH)
    row2d = jnp.concatenate([row, pad_rows]).reshape(NW * CPW, CH)
    val2d = jnp.pad(val, (0, pad)).reshape(NW * CPW, CH)

    embeds = jnp.concatenate([uEmbeds, iEmbeds], axis=0)
    embeds = jnp.pad(embeds, ((0, N_PAD - N), (0, 0)))
    acc = embeds
    cur = embeds
    for _ in range(N_LAYERS):
        parts = _spmm_layer(col2d, row2d, val2d, cur)
        cur, acc = _combine(parts, acc)
    return acc[:N_USER], acc[N_USER:N]
